# baseline (speedup 1.0000x reference)
import functools

import jax
import jax.numpy as jnp
from jax import lax
from jax.experimental import pallas as pl
from jax.experimental.pallas import tpu as pltpu
from jax.experimental.pallas import tpu_sc as plsc

_L = 16

_SHAPE_RATIO_THR = 0.25
_AREA_RATIO_THR = 0.01
_OBJECTNESS_THR = 0.5
_NMS_THR = 0.3


def _nms_body(n_pad, x1h, y1h, x2h, y2h, sh, imgh,
              ox1h, oy1h, ox2h, oy2h, osh,
              x1, y1, x2, y2, sc, area, sup,
              ox1, oy1, ox2, oy2, osc, img):
    nch = n_pad // _L
    cid = lax.axis_index("c")
    sid = lax.axis_index("s")

    @pl.when((cid == 0) & (sid == 0))
    def _():
        pltpu.sync_copy(x1h, x1)
        pltpu.sync_copy(y1h, y1)
        pltpu.sync_copy(x2h, x2)
        pltpu.sync_copy(y2h, y2)
        pltpu.sync_copy(sh, sc)
        pltpu.sync_copy(imgh, img)

        ix1 = img[0]
        iy1 = img[1]
        ix2 = img[2]
        iy2 = img[3]
        aimg = (ix2 - ix1) * (iy2 - iy1)

        def filt(c, acc):
            b = c * _L
            dx1 = x1[pl.ds(b, _L)]
            dy1 = y1[pl.ds(b, _L)]
            dx2 = x2[pl.ds(b, _L)]
            dy2 = y2[pl.ds(b, _L)]
            ds = sc[pl.ds(b, _L)]
            w = dx2 - dx1
            h = dy2 - dy1
            srat = w / (h + 1e-12)
            vshape = (srat > _SHAPE_RATIO_THR) & (srat < 1.0 / _SHAPE_RATIO_THR)
            iw = jnp.maximum(jnp.minimum(ix2, dx2) - jnp.maximum(ix1, dx1), 0.0)
            ih = jnp.maximum(jnp.minimum(iy2, dy2) - jnp.maximum(iy1, dy1), 0.0)
            iof = iw * ih / (aimg + 1e-6)
            v = vshape & (iof > _AREA_RATIO_THR) & (ds > _OBJECTNESS_THR)
            area[pl.ds(b, _L)] = w * h
            sup[pl.ds(b, _L)] = jnp.where(v, 0, 1).astype(jnp.int32)
            return acc + jnp.where(v, 1, 0).astype(jnp.int32)

        acc = lax.fori_loop(0, nch, filt, jnp.zeros((_L,), jnp.int32))
        nvalid = acc[0]
        for _i in range(1, _L):
            nvalid = nvalid + acc[_i]

        @pl.when(nvalid == 0)
        def _():
            lane = lax.iota(jnp.int32, _L)
            sup[pl.ds(0, _L)] = jnp.where(lane == 0, 0, 1).astype(jnp.int32)

        def outer(co, carry):
            b0 = co * _L
            cx1 = x1[pl.ds(b0, _L)]
            cy1 = y1[pl.ds(b0, _L)]
            cx2 = x2[pl.ds(b0, _L)]
            cy2 = y2[pl.ds(b0, _L)]
            car = area[pl.ds(b0, _L)]
            for lane in range(_L):
                k = b0 + lane
                supk = sup[pl.ds(b0, _L)][lane]

                @pl.when(supk == 0)
                def _(lane=lane, k=k):
                    bx1 = cx1[lane]
                    by1 = cy1[lane]
                    bx2 = cx2[lane]
                    by2 = cy2[lane]
                    ake = car[lane] + 1e-6

                    iw0 = jnp.maximum(jnp.minimum(bx2, cx2) - jnp.maximum(bx1, cx1), 0.0)
                    ih0 = jnp.maximum(jnp.minimum(by2, cy2) - jnp.maximum(by1, cy1), 0.0)
                    it0 = iw0 * ih0
                    hit0 = (it0 > (ake + car - it0) * _NMS_THR) & (lax.iota(jnp.int32, _L) > lane)
                    sv0 = sup[pl.ds(b0, _L)]
                    sup[pl.ds(b0, _L)] = sv0 | jnp.where(hit0, 1, 0).astype(jnp.int32)

                    def inner(c, icarry):
                        b = c * _L
                        dx1 = x1[pl.ds(b, _L)]
                        dy1 = y1[pl.ds(b, _L)]
                        dx2 = x2[pl.ds(b, _L)]
                        dy2 = y2[pl.ds(b, _L)]
                        da = area[pl.ds(b, _L)]
                        iw = jnp.maximum(jnp.minimum(bx2, dx2) - jnp.maximum(bx1, dx1), 0.0)
                        ih = jnp.maximum(jnp.minimum(by2, dy2) - jnp.maximum(by1, dy1), 0.0)
                        inter = iw * ih
                        hit = inter > (ake + da - inter) * _NMS_THR
                        sv = sup[pl.ds(b, _L)]
                        sup[pl.ds(b, _L)] = sv | jnp.where(hit, 1, 0).astype(jnp.int32)
                        return icarry

                    lax.fori_loop(co + 1, nch, inner, jnp.int32(0))

            return carry

        lax.fori_loop(0, nch, outer, jnp.int32(0))

        def outp(c, carry):
            b = c * _L
            kf = jnp.where(sup[pl.ds(b, _L)] == 0, 1.0, 0.0)
            ox1[pl.ds(b, _L)] = x1[pl.ds(b, _L)] * kf
            oy1[pl.ds(b, _L)] = y1[pl.ds(b, _L)] * kf
            ox2[pl.ds(b, _L)] = x2[pl.ds(b, _L)] * kf
            oy2[pl.ds(b, _L)] = y2[pl.ds(b, _L)] * kf
            osc[pl.ds(b, _L)] = sc[pl.ds(b, _L)] * kf
            return carry

        lax.fori_loop(0, nch, outp, jnp.int32(0))

        pltpu.sync_copy(ox1, ox1h)
        pltpu.sync_copy(oy1, oy1h)
        pltpu.sync_copy(ox2, ox2h)
        pltpu.sync_copy(oy2, oy2h)
        pltpu.sync_copy(osc, osh)


@functools.partial(jax.jit, static_argnums=(6,))
def _run(x1, y1, x2, y2, ss, img, n_pad):
    mesh = plsc.VectorSubcoreMesh(core_axis_name="c", subcore_axis_name="s")
    f32 = jnp.float32
    out = jax.ShapeDtypeStruct((n_pad,), f32)
    vec = pltpu.VMEM((n_pad,), f32)
    kern = functools.partial(
        pl.kernel,
        mesh=mesh,
        out_type=[out, out, out, out, out],
        scratch_types=[
            vec, vec, vec, vec, vec,
            vec,
            pltpu.VMEM((n_pad,), jnp.int32),
            vec, vec, vec, vec, vec,
            pltpu.VMEM((4, _L), f32),
        ],
    )(functools.partial(_nms_body, n_pad))
    return kern(x1, y1, x2, y2, ss, img)


def kernel(boxes, scores, image_boxes):
    n = boxes.shape[0]
    n_pad = -(-n // 128) * 128
    pad = n_pad - n
    order = jnp.argsort(-scores, stable=True)
    bs = boxes[order]
    zs = jnp.zeros((pad,), jnp.float32)
    x1 = jnp.concatenate([bs[:, 0], zs])
    y1 = jnp.concatenate([bs[:, 1], zs])
    x2 = jnp.concatenate([bs[:, 2], zs])
    y2 = jnp.concatenate([bs[:, 3], zs])
    ss = jnp.concatenate([scores[order], zs])
    img = jnp.broadcast_to(image_boxes[0].reshape(4, 1), (4, _L)).astype(jnp.float32)
    ox1, oy1, ox2, oy2, osc = _run(x1, y1, x2, y2, ss, img, int(n_pad))
    rows = jnp.stack([ox1[:n], oy1[:n], ox2[:n], oy2[:n], osc[:n]], axis=1)
    return jnp.zeros((n, 5), jnp.float32).at[order].set(rows)


# speedup vs baseline: 1.3254x; 1.3254x over previous
import functools

import jax
import jax.numpy as jnp
from jax import lax
from jax.experimental import pallas as pl
from jax.experimental.pallas import tpu as pltpu
from jax.experimental.pallas import tpu_sc as plsc

_L = 16

_SHAPE_RATIO_THR = 0.25
_AREA_RATIO_THR = 0.01
_OBJECTNESS_THR = 0.5
_NMS_THR = 0.3


def _nms_body(n_pad, x1h, y1h, x2h, y2h, sh, imgh,
              ox1h, oy1h, ox2h, oy2h, osh,
              x1, y1, x2, y2, sc, area, sup,
              ox1, oy1, ox2, oy2, osc, img):
    nch = n_pad // _L
    cid = lax.axis_index("c")
    sid = lax.axis_index("s")

    @pl.when((cid == 0) & (sid == 0))
    def _():
        pltpu.sync_copy(x1h, x1)
        pltpu.sync_copy(y1h, y1)
        pltpu.sync_copy(x2h, x2)
        pltpu.sync_copy(y2h, y2)
        pltpu.sync_copy(sh, sc)
        pltpu.sync_copy(imgh, img)

        ix1 = img[0]
        iy1 = img[1]
        ix2 = img[2]
        iy2 = img[3]
        aimg = (ix2 - ix1) * (iy2 - iy1)

        def filt(c, acc):
            b = c * _L
            dx1 = x1[pl.ds(b, _L)]
            dy1 = y1[pl.ds(b, _L)]
            dx2 = x2[pl.ds(b, _L)]
            dy2 = y2[pl.ds(b, _L)]
            ds = sc[pl.ds(b, _L)]
            w = dx2 - dx1
            h = dy2 - dy1
            srat = w / (h + 1e-12)
            vshape = (srat > _SHAPE_RATIO_THR) & (srat < 1.0 / _SHAPE_RATIO_THR)
            iw = jnp.maximum(jnp.minimum(ix2, dx2) - jnp.maximum(ix1, dx1), 0.0)
            ih = jnp.maximum(jnp.minimum(iy2, dy2) - jnp.maximum(iy1, dy1), 0.0)
            iof = iw * ih / (aimg + 1e-6)
            v = vshape & (iof > _AREA_RATIO_THR) & (ds > _OBJECTNESS_THR)
            area[pl.ds(b, _L)] = w * h
            sup[pl.ds(b, _L)] = jnp.where(v, 0, 1).astype(jnp.int32)
            return acc + jnp.where(v, 1, 0).astype(jnp.int32)

        acc = lax.fori_loop(0, nch, filt, jnp.zeros((_L,), jnp.int32))
        nvalid = acc[0]
        for _i in range(1, _L):
            nvalid = nvalid + acc[_i]

        @pl.when(nvalid == 0)
        def _():
            lane = lax.iota(jnp.int32, _L)
            sup[pl.ds(0, _L)] = jnp.where(lane == 0, 0, 1).astype(jnp.int32)

        def outer(co, carry):
            b0 = co * _L
            cx1 = x1[pl.ds(b0, _L)]
            cy1 = y1[pl.ds(b0, _L)]
            cx2 = x2[pl.ds(b0, _L)]
            cy2 = y2[pl.ds(b0, _L)]
            car = area[pl.ds(b0, _L)]
            for lane in range(_L):
                k = b0 + lane
                supk = sup[pl.ds(b0, _L)][lane]

                @pl.when(supk == 0)
                def _(lane=lane, k=k):
                    bx1 = cx1[lane]
                    by1 = cy1[lane]
                    bx2 = cx2[lane]
                    by2 = cy2[lane]
                    ak = car[lane]

                    def inner(c, icarry):
                        b = c * _L
                        dx1 = x1[pl.ds(b, _L)]
                        dy1 = y1[pl.ds(b, _L)]
                        dx2 = x2[pl.ds(b, _L)]
                        dy2 = y2[pl.ds(b, _L)]
                        da = area[pl.ds(b, _L)]
                        iw = jnp.maximum(jnp.minimum(bx2, dx2) - jnp.maximum(bx1, dx1), 0.0)
                        ih = jnp.maximum(jnp.minimum(by2, dy2) - jnp.maximum(by1, dy1), 0.0)
                        inter = iw * ih
                        gidx = b + lax.iota(jnp.int32, _L)
                        hit = (inter > (ak + da - inter + 1e-6) * _NMS_THR) & (gidx > k)
                        sv = sup[pl.ds(b, _L)]
                        sup[pl.ds(b, _L)] = sv | jnp.where(hit, 1, 0).astype(jnp.int32)
                        return icarry

                    lax.fori_loop(co, nch, inner, jnp.int32(0))

            return carry

        lax.fori_loop(0, nch, outer, jnp.int32(0))

        def outp(c, carry):
            b = c * _L
            kf = jnp.where(sup[pl.ds(b, _L)] == 0, 1.0, 0.0)
            ox1[pl.ds(b, _L)] = x1[pl.ds(b, _L)] * kf
            oy1[pl.ds(b, _L)] = y1[pl.ds(b, _L)] * kf
            ox2[pl.ds(b, _L)] = x2[pl.ds(b, _L)] * kf
            oy2[pl.ds(b, _L)] = y2[pl.ds(b, _L)] * kf
            osc[pl.ds(b, _L)] = sc[pl.ds(b, _L)] * kf
            return carry

        lax.fori_loop(0, nch, outp, jnp.int32(0))

        pltpu.sync_copy(ox1, ox1h)
        pltpu.sync_copy(oy1, oy1h)
        pltpu.sync_copy(ox2, ox2h)
        pltpu.sync_copy(oy2, oy2h)
        pltpu.sync_copy(osc, osh)


@functools.partial(jax.jit, static_argnums=(6,))
def _run(x1, y1, x2, y2, ss, img, n_pad):
    mesh = plsc.VectorSubcoreMesh(core_axis_name="c", subcore_axis_name="s")
    f32 = jnp.float32
    out = jax.ShapeDtypeStruct((n_pad,), f32)
    vec = pltpu.VMEM((n_pad,), f32)
    kern = functools.partial(
        pl.kernel,
        mesh=mesh,
        out_type=[out, out, out, out, out],
        scratch_types=[
            vec, vec, vec, vec, vec,
            vec,
            pltpu.VMEM((n_pad,), jnp.int32),
            vec, vec, vec, vec, vec,
            pltpu.VMEM((4, _L), f32),
        ],
    )(functools.partial(_nms_body, n_pad))
    return kern(x1, y1, x2, y2, ss, img)


def kernel(boxes, scores, image_boxes):
    n = boxes.shape[0]
    n_pad = -(-n // 128) * 128
    pad = n_pad - n
    order = jnp.argsort(-scores, stable=True)
    bs = boxes[order]
    zs = jnp.zeros((pad,), jnp.float32)
    x1 = jnp.concatenate([bs[:, 0], zs])
    y1 = jnp.concatenate([bs[:, 1], zs])
    x2 = jnp.concatenate([bs[:, 2], zs])
    y2 = jnp.concatenate([bs[:, 3], zs])
    ss = jnp.concatenate([scores[order], zs])
    img = jnp.broadcast_to(image_boxes[0].reshape(4, 1), (4, _L)).astype(jnp.float32)
    ox1, oy1, ox2, oy2, osc = _run(x1, y1, x2, y2, ss, img, int(n_pad))
    rows = jnp.stack([ox1[:n], oy1[:n], ox2[:n], oy2[:n], osc[:n]], axis=1)
    return jnp.zeros((n, 5), jnp.float32).at[order].set(rows)
